# baseline (speedup 1.0000x reference)
import functools

import jax
import jax.numpy as jnp
from jax import lax
from jax.experimental import pallas as pl
from jax.experimental.pallas import tpu as pltpu
from jax.experimental.pallas import tpu_sc as plsc

_NC = 2
_NS = 16
_NW = _NC * _NS
_LANES = 16


def _sc_call(H, W, D):
    RPW = H // _NW
    CC = 32
    NCH = W // CC
    DH = D // (2 * _LANES)

    mesh = plsc.VectorSubcoreMesh(core_axis_name="c", subcore_axis_name="s")

    @functools.partial(
        pl.kernel,
        out_type=jax.ShapeDtypeStruct((H * W, D), jnp.float32),
        mesh=mesh,
        scratch_types=[
            pltpu.VMEM((RPW,), jnp.int32),
            pltpu.VMEM((CC,), jnp.int32),
            pltpu.VMEM((RPW, D), jnp.float32),
            pltpu.VMEM((CC, D), jnp.float32),
            pltpu.VMEM((CC, D), jnp.float32),
            pltpu.SemaphoreType.DMA,
        ],
    )
    def call(ridx_hbm, cidx_hbm, row_hbm, col_hbm, out_hbm,
             ridx_v, cidx_v, row_v, col_v, out_v, sem):
        wid = lax.axis_index("c") * _NS + lax.axis_index("s")
        base = wid * RPW
        pltpu.sync_copy(ridx_hbm.at[pl.ds(base * 1, RPW)], ridx_v)
        pltpu.async_copy(row_hbm.at[ridx_v], row_v, sem).wait()
        for c in range(NCH):
            pltpu.sync_copy(cidx_hbm.at[pl.ds(c * CC, CC)], cidx_v)
            pltpu.async_copy(col_hbm.at[cidx_v], col_v, sem).wait()

            def il_body(il, _):
                for half in range(2):
                    off = half * DH * _LANES
                    rvecs = [row_v[il, pl.ds(off + d * _LANES, _LANES)]
                             for d in range(DH)]

                    def j_body(j, _):
                        for d in range(DH):
                            sl = pl.ds(off + d * _LANES, _LANES)
                            out_v[j, sl] = col_v[j, sl] + rvecs[d]
                        return ()

                    lax.fori_loop(0, CC, j_body, ())
                start = (base + il) * W + c * CC
                pltpu.sync_copy(out_v, out_hbm.at[pl.ds(start, CC)])
                return ()

            lax.fori_loop(0, RPW, il_body, ())

    return call


def kernel(h, w, row_embed, col_embed):
    H, D = row_embed.shape
    W, _ = col_embed.shape
    ridx = jnp.minimum(jnp.arange(H, dtype=jnp.int32),
                       jnp.asarray(h, jnp.int32) - 1)
    cidx = jnp.minimum(jnp.arange(W, dtype=jnp.int32),
                       jnp.asarray(w, jnp.int32) - 1)
    return _sc_call(H, W, D)(ridx, cidx, row_embed, col_embed)


# speedup vs baseline: 1.2955x; 1.2955x over previous
import functools

import jax
import jax.numpy as jnp
from jax import lax
from jax.experimental import pallas as pl
from jax.experimental.pallas import tpu as pltpu
from jax.experimental.pallas import tpu_sc as plsc

_NC = 2
_NS = 16
_NW = _NC * _NS
_LANES = 16


def _sc_call(H, W, D):
    RPW = H // _NW
    CC = 32
    NCH = W // CC
    DH = D // (2 * _LANES)

    mesh = plsc.VectorSubcoreMesh(core_axis_name="c", subcore_axis_name="s")

    @functools.partial(
        pl.kernel,
        out_type=jax.ShapeDtypeStruct((H * W, D), jnp.float32),
        mesh=mesh,
        scratch_types=[
            pltpu.VMEM((RPW,), jnp.int32),
            pltpu.VMEM((CC,), jnp.int32),
            pltpu.VMEM((RPW, D), jnp.float32),
            pltpu.VMEM((CC, D), jnp.float32),
            pltpu.VMEM((CC, D), jnp.float32),
            pltpu.VMEM((CC, D), jnp.float32),
            pltpu.SemaphoreType.DMA,
            pltpu.SemaphoreType.DMA,
            pltpu.SemaphoreType.DMA,
        ],
    )
    def call(ridx_hbm, cidx_hbm, row_hbm, col_hbm, out_hbm,
             ridx_v, cidx_v, row_v, col_v, out_v0, out_v1,
             gsem, sem0, sem1):
        wid = lax.axis_index("c") * _NS + lax.axis_index("s")
        base = wid * RPW
        out_bufs = (out_v0, out_v1)
        sems = (sem0, sem1)
        pltpu.sync_copy(ridx_hbm.at[pl.ds(base * 1, RPW)], ridx_v)
        pltpu.async_copy(row_hbm.at[ridx_v], row_v, gsem).wait()
        for c in range(NCH):
            pltpu.sync_copy(cidx_hbm.at[pl.ds(c * CC, CC)], cidx_v)
            pltpu.async_copy(col_hbm.at[cidx_v], col_v, gsem).wait()

            def il2_body(il2, _, c=c):
                for b in range(2):
                    il = il2 * 2 + b
                    buf = out_bufs[b]
                    sem = sems[b]
                    start = (base + il) * W + c * CC
                    dst = out_hbm.at[pl.ds(start, CC)]
                    if c == 0:
                        @pl.when(il2 > 0)
                        def _():
                            pltpu.make_async_copy(buf, dst, sem).wait()
                    else:
                        pltpu.make_async_copy(buf, dst, sem).wait()
                    for half in range(2):
                        off = half * DH * _LANES
                        rvecs = [row_v[il, pl.ds(off + d * _LANES, _LANES)]
                                 for d in range(DH)]

                        def j_body(j, _, buf=buf, off=off, rvecs=rvecs):
                            for d in range(DH):
                                sl = pl.ds(off + d * _LANES, _LANES)
                                buf[j, sl] = col_v[j, sl] + rvecs[d]
                            return ()

                        lax.fori_loop(0, CC, j_body, ())
                    pltpu.async_copy(buf, dst, sem)
                return ()

            lax.fori_loop(0, RPW // 2, il2_body, ())
        for b in range(2):
            pltpu.make_async_copy(
                out_bufs[b], out_hbm.at[pl.ds(base * W, CC)], sems[b]).wait()

    return call


def kernel(h, w, row_embed, col_embed):
    H, D = row_embed.shape
    W, _ = col_embed.shape
    ridx = jnp.minimum(jnp.arange(H, dtype=jnp.int32),
                       jnp.asarray(h, jnp.int32) - 1)
    cidx = jnp.minimum(jnp.arange(W, dtype=jnp.int32),
                       jnp.asarray(w, jnp.int32) - 1)
    return _sc_call(H, W, D)(ridx, cidx, row_embed, col_embed)
